# baseline (speedup 1.0000x reference)
import jax
import jax.numpy as jnp
from jax.experimental import pallas as pl
from jax.experimental.pallas import tpu as pltpu

NODE = 256
REL = 4
QCOL = NODE * REL
BLK_J = 16
BLK_R = BLK_J * NODE
NSTEP = NODE // 2 // BLK_J
HALF_J = NODE // 2


def _contract(colblk, fcb):
    partial = jnp.zeros((1, NODE), jnp.float32)
    for jl in range(BLK_J):
        prod = colblk[:, jl:jl + 1] * fcb[jl * NODE:(jl + 1) * NODE, :]
        partial = partial + jnp.sum(prod, axis=0, keepdims=True)
    return partial


def _colsel(outT, j0):
    ji = jax.lax.broadcasted_iota(jnp.int32, (NODE, BLK_J), 0)
    ti = jax.lax.broadcasted_iota(jnp.int32, (NODE, BLK_J), 1)
    sel = jnp.where(ji == j0 + ti, 1.0, 0.0)
    return jnp.dot(outT, sel, preferred_element_type=jnp.float32)


def _body(xm_ref, wt_ref, bcol_ref, fca_ref, fcb_ref, fcbias_ref,
          out_ref, outT_s):
    step = pl.program_id(0)

    @pl.when(step == 0)
    def _gcn():
        m = (xm_ref[...] != 0.0).astype(jnp.float32)
        qi = jax.lax.broadcasted_iota(jnp.int32, (QCOL, NODE), 0)
        ji = jax.lax.broadcasted_iota(jnp.int32, (QCOL, NODE), 1)
        sel = jnp.where((qi // REL) == ji, 1.0, 0.0)
        c = jnp.dot(m, sel, preferred_element_type=jnp.float32)
        deg = 1.0 + jnp.sum(c, axis=0, keepdims=True)
        dis = jax.lax.rsqrt(deg)
        wt = wt_ref[...]
        tT = jnp.dot(wt * dis, c, preferred_element_type=jnp.float32)
        outT_s[...] = dis * tT + (dis * dis) * wt + bcol_ref[...]

    outT = outT_s[...]
    pa = _contract(_colsel(outT, step * BLK_J), fca_ref[...])
    pb = _contract(_colsel(outT, HALF_J + step * BLK_J), fcb_ref[...])
    partial = pa + pb

    @pl.when(step == 0)
    def _init():
        out_ref[...] = partial + fcbias_ref[...]

    @pl.when(step > 0)
    def _acc():
        out_ref[...] = out_ref[...] + partial


def kernel(x, W, b_gcn, fc_W, fc_b):
    xm = x.reshape(NODE, QCOL)
    wt = W.T
    bcol = b_gcn.reshape(NODE, 1)
    fcbias = fc_b.reshape(1, NODE)
    fc3 = fc_W.reshape(2, HALF_J * NODE, NODE)
    fca = fc3[0]
    fcb = fc3[1]
    return pl.pallas_call(
        _body,
        grid=(NSTEP,),
        in_specs=[
            pl.BlockSpec((NODE, QCOL), lambda s: (0, 0)),
            pl.BlockSpec((NODE, NODE), lambda s: (0, 0)),
            pl.BlockSpec((NODE, 1), lambda s: (0, 0)),
            pl.BlockSpec((BLK_R, NODE), lambda s: (s, 0)),
            pl.BlockSpec((BLK_R, NODE), lambda s: (s, 0)),
            pl.BlockSpec((1, NODE), lambda s: (0, 0)),
        ],
        out_specs=pl.BlockSpec((1, NODE), lambda s: (0, 0)),
        out_shape=jax.ShapeDtypeStruct((1, NODE), jnp.float32),
        scratch_shapes=[pltpu.VMEM((NODE, NODE), jnp.float32)],
    )(xm, wt, bcol, fca, fcb, fcbias)


# speedup vs baseline: 2.0942x; 2.0942x over previous
import jax
import jax.numpy as jnp
from jax.experimental import pallas as pl
from jax.experimental.pallas import tpu as pltpu

NODE = 256
REL = 4
QCOL = NODE * REL
BLK_J = 16
BLK_R = BLK_J * NODE
NSTEP = NODE // 2 // BLK_J
HALF_J = NODE // 2


def _contract(colblk, fcb):
    partial = jnp.zeros((1, NODE), jnp.float32)
    for jl in range(BLK_J):
        prod = colblk[:, jl:jl + 1] * fcb[jl * NODE:(jl + 1) * NODE, :]
        partial = partial + jnp.sum(prod, axis=0, keepdims=True)
    return partial


def _colsel(outT, j0):
    ji = jax.lax.broadcasted_iota(jnp.int32, (NODE, BLK_J), 0)
    ti = jax.lax.broadcasted_iota(jnp.int32, (NODE, BLK_J), 1)
    sel = jnp.where(ji == j0 + ti, 1.0, 0.0)
    return jnp.dot(outT, sel, preferred_element_type=jnp.float32)


def _body(xm_ref, wt_ref, bcol_ref, fca_ref, fcb_ref, fcbias_ref,
          out_ref, outT_s):
    step = pl.program_id(0)

    @pl.when(step == 0)
    def _gcn():
        m = (xm_ref[...] != 0.0).astype(jnp.float32)
        qi = jax.lax.broadcasted_iota(jnp.int32, (QCOL, NODE), 0)
        ji = jax.lax.broadcasted_iota(jnp.int32, (QCOL, NODE), 1)
        sel = jnp.where((qi // REL) == ji, 1.0, 0.0)
        c = jnp.dot(m, sel, preferred_element_type=jnp.float32)
        deg = 1.0 + jnp.sum(c, axis=0, keepdims=True)
        dis = jax.lax.rsqrt(deg)
        wt = wt_ref[...]
        tT = jnp.dot(wt * dis, c, preferred_element_type=jnp.float32)
        outT_s[...] = dis * tT + (dis * dis) * wt + bcol_ref[...]

    outT = outT_s[...]
    pa = _contract(_colsel(outT, step * BLK_J), fca_ref[...])
    pb = _contract(_colsel(outT, HALF_J + step * BLK_J), fcb_ref[...])
    partial = pa + pb

    @pl.when(step == 0)
    def _init():
        out_ref[...] = partial + fcbias_ref[...]

    @pl.when(step > 0)
    def _acc():
        out_ref[...] = out_ref[...] + partial


def kernel(x, W, b_gcn, fc_W, fc_b):
    xm = x.reshape(NODE, QCOL)
    wt = W.T
    bcol = b_gcn.reshape(NODE, 1)
    fcbias = fc_b.reshape(1, NODE)
    return pl.pallas_call(
        _body,
        grid=(NSTEP,),
        in_specs=[
            pl.BlockSpec((NODE, QCOL), lambda s: (0, 0)),
            pl.BlockSpec((NODE, NODE), lambda s: (0, 0)),
            pl.BlockSpec((NODE, 1), lambda s: (0, 0)),
            pl.BlockSpec((BLK_R, NODE), lambda s: (s, 0)),
            pl.BlockSpec((BLK_R, NODE), lambda s: (s + NSTEP, 0)),
            pl.BlockSpec((1, NODE), lambda s: (0, 0)),
        ],
        out_specs=pl.BlockSpec((1, NODE), lambda s: (0, 0)),
        out_shape=jax.ShapeDtypeStruct((1, NODE), jnp.float32),
        scratch_shapes=[pltpu.VMEM((NODE, NODE), jnp.float32)],
    )(xm, wt, bcol, fc_W, fc_W, fcbias)


# speedup vs baseline: 2.2123x; 1.0564x over previous
import jax
import jax.numpy as jnp
from jax.experimental import pallas as pl
from jax.experimental.pallas import tpu as pltpu

NODE = 256
REL = 4
QCOL = NODE * REL
BLK_J = 32
BLK_R = BLK_J * NODE
NSTEP = NODE // BLK_J


def _body(xm_ref, wt_ref, bcol_ref, fcb_ref, fcbias_ref, out_ref, outT_s):
    step = pl.program_id(0)

    @pl.when(step == 0)
    def _gcn():
        m = (xm_ref[...] != 0.0).astype(jnp.float32)
        qi = jax.lax.broadcasted_iota(jnp.int32, (QCOL, NODE), 0)
        ji = jax.lax.broadcasted_iota(jnp.int32, (QCOL, NODE), 1)
        sel = jnp.where((qi // REL) == ji, 1.0, 0.0)
        c = jnp.dot(m.astype(jnp.bfloat16), sel.astype(jnp.bfloat16),
                    preferred_element_type=jnp.float32)
        deg = 1.0 + jnp.sum(c, axis=0, keepdims=True)
        dis = jax.lax.rsqrt(deg)
        wt = wt_ref[...]
        tT = jnp.dot(wt * dis, c, preferred_element_type=jnp.float32)
        outT_s[...] = dis * tT + (dis * dis) * wt + bcol_ref[...]

    ji2 = jax.lax.broadcasted_iota(jnp.int32, (NODE, BLK_J), 0)
    ti = jax.lax.broadcasted_iota(jnp.int32, (NODE, BLK_J), 1)
    sel_e = jnp.where(ji2 == step * BLK_J + ti, 1.0, 0.0)
    colblk = jnp.dot(outT_s[...], sel_e,
                     preferred_element_type=jnp.float32)

    fcb = fcb_ref[...]
    partial = jnp.zeros((1, NODE), jnp.float32)
    for jl in range(BLK_J):
        prod = colblk[:, jl:jl + 1] * fcb[jl * NODE:(jl + 1) * NODE, :]
        partial = partial + jnp.sum(prod, axis=0, keepdims=True)

    @pl.when(step == 0)
    def _init():
        out_ref[...] = partial + fcbias_ref[...]

    @pl.when(step > 0)
    def _acc():
        out_ref[...] = out_ref[...] + partial


def kernel(x, W, b_gcn, fc_W, fc_b):
    xm = x.reshape(NODE, QCOL)
    wt = W.T
    bcol = b_gcn.reshape(NODE, 1)
    fcbias = fc_b.reshape(1, NODE)
    return pl.pallas_call(
        _body,
        grid=(NSTEP,),
        in_specs=[
            pl.BlockSpec((NODE, QCOL), lambda s: (0, 0)),
            pl.BlockSpec((NODE, NODE), lambda s: (0, 0)),
            pl.BlockSpec((NODE, 1), lambda s: (0, 0)),
            pl.BlockSpec((BLK_R, NODE), lambda s: (s, 0)),
            pl.BlockSpec((1, NODE), lambda s: (0, 0)),
        ],
        out_specs=pl.BlockSpec((1, NODE), lambda s: (0, 0)),
        out_shape=jax.ShapeDtypeStruct((1, NODE), jnp.float32),
        scratch_shapes=[pltpu.VMEM((NODE, NODE), jnp.float32)],
    )(xm, wt, bcol, fc_W, fcbias)


# speedup vs baseline: 2.2414x; 1.0131x over previous
import jax
import jax.numpy as jnp
from jax.experimental import pallas as pl
from jax.experimental.pallas import tpu as pltpu

NODE = 256
REL = 4
QCOL = NODE * REL
BLK_J = 32
BLK_R = BLK_J * NODE
NSTEP = NODE // BLK_J


def _body(xm_ref, wt_ref, bcol_ref, fcb_ref, fcbias_ref, out_ref, outT3_s):
    step = pl.program_id(0)

    @pl.when(step == 0)
    def _gcn():
        m = (xm_ref[...] != 0.0).astype(jnp.float32)
        qi = jax.lax.broadcasted_iota(jnp.int32, (QCOL, NODE), 0)
        ji = jax.lax.broadcasted_iota(jnp.int32, (QCOL, NODE), 1)
        sel = jnp.where((qi // REL) == ji, 1.0, 0.0)
        c = jnp.dot(m.astype(jnp.bfloat16), sel.astype(jnp.bfloat16),
                    preferred_element_type=jnp.float32)
        deg = 1.0 + jnp.sum(c, axis=0, keepdims=True)
        dis = jax.lax.rsqrt(deg)
        wt = wt_ref[...]
        tT = jnp.dot(wt * dis, c, preferred_element_type=jnp.float32)
        outT = dis * tT + (dis * dis) * wt + bcol_ref[...]
        for s_ in range(NSTEP):
            outT3_s[s_] = outT[:, s_ * BLK_J:(s_ + 1) * BLK_J]

    colblk = outT3_s[step]

    fcb = fcb_ref[...]
    partial = jnp.zeros((1, NODE), jnp.float32)
    for jl in range(BLK_J):
        prod = colblk[:, jl:jl + 1] * fcb[jl * NODE:(jl + 1) * NODE, :]
        partial = partial + jnp.sum(prod, axis=0, keepdims=True)

    @pl.when(step == 0)
    def _init():
        out_ref[...] = partial + fcbias_ref[...]

    @pl.when(step > 0)
    def _acc():
        out_ref[...] = out_ref[...] + partial


def kernel(x, W, b_gcn, fc_W, fc_b):
    xm = x.reshape(NODE, QCOL)
    wt = W.T
    bcol = b_gcn.reshape(NODE, 1)
    fcbias = fc_b.reshape(1, NODE)
    return pl.pallas_call(
        _body,
        grid=(NSTEP,),
        in_specs=[
            pl.BlockSpec((NODE, QCOL), lambda s: (0, 0)),
            pl.BlockSpec((NODE, NODE), lambda s: (0, 0)),
            pl.BlockSpec((NODE, 1), lambda s: (0, 0)),
            pl.BlockSpec((BLK_R, NODE), lambda s: (s, 0)),
            pl.BlockSpec((1, NODE), lambda s: (0, 0)),
        ],
        out_specs=pl.BlockSpec((1, NODE), lambda s: (0, 0)),
        out_shape=jax.ShapeDtypeStruct((1, NODE), jnp.float32),
        scratch_shapes=[pltpu.VMEM((NSTEP, NODE, BLK_J), jnp.float32)],
    )(xm, wt, bcol, fc_W, fcbias)


# speedup vs baseline: 2.9745x; 1.3271x over previous
import jax
import jax.numpy as jnp
from jax.experimental import pallas as pl
from jax.experimental.pallas import tpu as pltpu

NODE = 256
REL = 4
QCOL = NODE * REL
BLK_J = 32
BLK_R = BLK_J * NODE
NSTEP = NODE // BLK_J


def _body(xm_ref, wt_ref, bcol_ref, fcb_ref, fcbias_ref, out_ref, outT3_s):
    step = pl.program_id(0)

    @pl.when(step == 0)
    def _gcn():
        m = (xm_ref[0] != 0.0).astype(jnp.float32)
        qi = jax.lax.broadcasted_iota(jnp.int32, (QCOL, NODE), 0)
        ji = jax.lax.broadcasted_iota(jnp.int32, (QCOL, NODE), 1)
        sel = jnp.where((qi // REL) == ji, 1.0, 0.0)
        c = jnp.dot(m.astype(jnp.bfloat16), sel.astype(jnp.bfloat16),
                    preferred_element_type=jnp.float32)
        deg = 1.0 + jnp.sum(c, axis=0, keepdims=True)
        dis = jax.lax.rsqrt(deg)
        wt = wt_ref[...]
        tT = jnp.dot(wt * dis, c, preferred_element_type=jnp.float32)
        outT = dis * tT + (dis * dis) * wt + bcol_ref[...]
        for s_ in range(NSTEP):
            outT3_s[s_] = outT[:, s_ * BLK_J:(s_ + 1) * BLK_J]

    colblk = outT3_s[step]

    fcb = fcb_ref[...]
    partial = jnp.zeros((1, NODE), jnp.float32)
    for jl in range(BLK_J):
        prod = colblk[:, jl:jl + 1] * fcb[jl * NODE:(jl + 1) * NODE, :]
        partial = partial + jnp.sum(prod, axis=0, keepdims=True)

    @pl.when(step == 0)
    def _init():
        out_ref[...] = partial + fcbias_ref[...]

    @pl.when(step > 0)
    def _acc():
        out_ref[...] = out_ref[...] + partial


def kernel(x, W, b_gcn, fc_W, fc_b):
    xm = x.reshape(1, NODE, QCOL)
    wt = W.T
    bcol = b_gcn.reshape(NODE, 1)
    fcbias = fc_b.reshape(1, NODE)
    return pl.pallas_call(
        _body,
        grid=(NSTEP,),
        in_specs=[
            pl.BlockSpec((1, NODE, QCOL), lambda s: (0, 0, 0)),
            pl.BlockSpec((NODE, NODE), lambda s: (0, 0)),
            pl.BlockSpec((NODE, 1), lambda s: (0, 0)),
            pl.BlockSpec((BLK_R, NODE), lambda s: (s, 0)),
            pl.BlockSpec((1, NODE), lambda s: (0, 0)),
        ],
        out_specs=pl.BlockSpec((1, NODE), lambda s: (0, 0)),
        out_shape=jax.ShapeDtypeStruct((1, NODE), jnp.float32),
        scratch_shapes=[pltpu.VMEM((NSTEP, NODE, BLK_J), jnp.float32)],
    )(xm, wt, bcol, fc_W, fcbias)


# speedup vs baseline: 3.3190x; 1.1158x over previous
import jax
import jax.numpy as jnp
from jax.experimental import pallas as pl
from jax.experimental.pallas import tpu as pltpu

NODE = 256
REL = 4
QCOL = NODE * REL
BLK_J = 32
BLK_R = BLK_J * NODE
NSTEP = NODE // BLK_J


def _body(xm_ref, w_ref, brow_ref, fcb_ref, fcbias_ref, out_ref, outT3_s):
    step = pl.program_id(0)

    @pl.when(step == 0)
    def _gcn():
        m = (xm_ref[0] != 0.0).astype(jnp.float32)
        qi = jax.lax.broadcasted_iota(jnp.int32, (QCOL, NODE), 0)
        ji = jax.lax.broadcasted_iota(jnp.int32, (QCOL, NODE), 1)
        sel = jnp.where((qi // REL) == ji, 1.0, 0.0)
        c = jnp.dot(m.astype(jnp.bfloat16), sel.astype(jnp.bfloat16),
                    preferred_element_type=jnp.float32)
        deg_row = 1.0 + jnp.sum(c, axis=0, keepdims=True)
        dis_row = jax.lax.rsqrt(deg_row)
        ones_col = jnp.full((NODE, 1), 1.0, jnp.float32)
        deg_col = 1.0 + jax.lax.dot_general(
            c, ones_col, (((0,), (0,)), ((), ())),
            preferred_element_type=jnp.float32)
        dis_col = jax.lax.rsqrt(deg_col)
        ii = jax.lax.broadcasted_iota(jnp.int32, (NODE, NODE), 0)
        jj = jax.lax.broadcasted_iota(jnp.int32, (NODE, NODE), 1)
        eye = jnp.where(ii == jj, 1.0, 0.0)
        g = (dis_col * c + eye * dis_row) * dis_row
        ones_row = jnp.full((1, NODE), 1.0, jnp.float32)
        w_aug = jnp.concatenate([w_ref[...], brow_ref[...]], axis=0)
        g_aug = jnp.concatenate([g, ones_row], axis=0)
        outT = jax.lax.dot_general(
            w_aug, g_aug, (((0,), (0,)), ((), ())),
            preferred_element_type=jnp.float32)
        for s_ in range(NSTEP):
            outT3_s[s_] = outT[:, s_ * BLK_J:(s_ + 1) * BLK_J]

    colblk = outT3_s[step]

    fcb = fcb_ref[...]
    partial = jnp.zeros((1, NODE), jnp.float32)
    for jl in range(BLK_J):
        prod = colblk[:, jl:jl + 1] * fcb[jl * NODE:(jl + 1) * NODE, :]
        partial = partial + jnp.sum(prod, axis=0, keepdims=True)

    @pl.when(step == 0)
    def _init():
        out_ref[...] = partial + fcbias_ref[...]

    @pl.when(step > 0)
    def _acc():
        out_ref[...] = out_ref[...] + partial


def kernel(x, W, b_gcn, fc_W, fc_b):
    xm = x.reshape(1, NODE, QCOL)
    brow = b_gcn.reshape(1, NODE)
    fcbias = fc_b.reshape(1, NODE)
    return pl.pallas_call(
        _body,
        grid=(NSTEP,),
        in_specs=[
            pl.BlockSpec((1, NODE, QCOL), lambda s: (0, 0, 0)),
            pl.BlockSpec((NODE, NODE), lambda s: (0, 0)),
            pl.BlockSpec((1, NODE), lambda s: (0, 0)),
            pl.BlockSpec((BLK_R, NODE), lambda s: (s, 0)),
            pl.BlockSpec((1, NODE), lambda s: (0, 0)),
        ],
        out_specs=pl.BlockSpec((1, NODE), lambda s: (0, 0)),
        out_shape=jax.ShapeDtypeStruct((1, NODE), jnp.float32),
        scratch_shapes=[pltpu.VMEM((NSTEP, NODE, BLK_J), jnp.float32)],
    )(xm, W, brow, fc_W, fcbias)
